# baseline (speedup 1.0000x reference)
import functools

import jax
import jax.numpy as jnp
from jax import lax
from jax.experimental import pallas as pl
from jax.experimental.pallas import tpu as pltpu
from jax.experimental.pallas import tpu_sc as plsc

NC = 2
NS = 16
CHUNK = 128
DEGW = 128
SPLIT_SLOW = 0
SPLIT_F = 0.26


def _mm_body(x_ref, w_ref, o_ref):
    o_ref[...] = jnp.dot(x_ref[...], w_ref[...],
                         preferred_element_type=jnp.float32)


def _scale_body(dp0_ref, dp1_ref, h1_ref, dis_ref, g_ref):
    deg = dp0_ref[:, 0:1] + dp1_ref[:, 0:1] + 1.0
    dis = lax.rsqrt(deg)
    dis_ref[...] = dis
    g_ref[...] = dis * h1_ref[...]


def _mid_body(s0_ref, s1_ref, g1_ref, dis_ref, w2_ref, b1_ref, g2_ref):
    st = s0_ref[...] + s1_ref[...] + g1_ref[...]
    conv = dis_ref[...] * st + b1_ref[...]
    a = jnp.maximum(conv, 0.0)
    h2 = jnp.dot(a, w2_ref[...], preferred_element_type=jnp.float32)
    g2_ref[...] = dis_ref[...] * h2


def _out_body(s0_ref, s1_ref, g2_ref, dis_ref, b2_ref, o_ref):
    st = s0_ref[...] + s1_ref[...] + g2_ref[...]
    conv = dis_ref[...] * st + b2_ref[...]
    m = jnp.max(conv, axis=1, keepdims=True)
    e = jnp.exp(conv - m)
    z = jnp.sum(e, axis=1, keepdims=True)
    o_ref[...] = conv - m - jnp.log(z)


def _tc(body, out_shape, *args):
    return pl.pallas_call(body, out_shape=out_shape)(*args)


def _sc_deg_body(nchunks, rpt, grp, col_hbm, ones_hbm, z_hbm, dp_out,
                 colv, ones_v, deg_sp, sems):
    c = lax.axis_index("c")
    s = lax.axis_index("s")
    pltpu.sync_copy(col_hbm.at[c, s], colv)
    pltpu.sync_copy(ones_hbm, ones_v)
    pltpu.sync_copy(z_hbm, deg_sp.at[pl.ds(s * rpt, rpt)])
    plsc.subcore_barrier()

    def group(gi, carry):
        descs = [pltpu.async_copy(ones_v, deg_sp.at[colv.at[gi * grp + b]],
                                  sems, add=True) for b in range(grp)]
        for desc in descs:
            desc.wait()
        return carry

    lax.fori_loop(0, nchunks // grp, group, 0)
    plsc.subcore_barrier()
    pltpu.sync_copy(deg_sp.at[pl.ds(s * rpt, rpt)], dp_out.at[c, s])


def _sc_prop_body(kcs, rpt, grp, g_hbm, row_hbm, col_hbm, z_hbm, sp_out,
                  rowv, colv, gbuf0, gbuf1, acc_sp, semg, sems):
    c = lax.axis_index("c")
    s = lax.axis_index("s")
    pltpu.sync_copy(z_hbm, acc_sp.at[pl.ds(s * rpt, rpt)])
    plsc.subcore_barrier()
    bufs = (gbuf0, gbuf1)

    def group(gi, carry):
        pltpu.sync_copy(row_hbm.at[c, s, pl.ds(gi * grp, grp)], rowv)
        pltpu.sync_copy(col_hbm.at[c, s, pl.ds(gi * grp, grp)], colv)
        g_desc = pltpu.async_copy(g_hbm.at[rowv.at[0]], bufs[0], semg)
        s_desc = None
        for b in range(grp):
            g_desc.wait()
            if s_desc is not None:
                s_desc.wait()
            if b + 1 < grp:
                g_desc = pltpu.async_copy(g_hbm.at[rowv.at[b + 1]],
                                          bufs[(b + 1) % 2], semg)
            s_desc = pltpu.async_copy(bufs[b % 2], acc_sp.at[colv.at[b]],
                                      sems, add=True)
        s_desc.wait()
        return carry

    ngrps = jnp.where(c == 0, kcs[0] // grp, kcs[1] // grp)
    lax.fori_loop(0, ngrps, group, 0)
    plsc.subcore_barrier()
    pltpu.sync_copy(acc_sp.at[pl.ds(s * rpt, rpt)], sp_out.at[c, s])


@jax.jit
def kernel(x, edge_index, W1, b1, W2, b2):
    n, d = x.shape
    e = edge_index.shape[1]
    e2 = 2 * e
    grp = 16
    k = -(-e2 // (NC * NS * CHUNK * grp)) * grp
    pade = NC * NS * k * CHUNK
    np_rows = -(-(n + 1) // (NS * 8)) * (NS * 8)
    rpt = np_rows // NS

    ei = edge_index.astype(jnp.int32)
    row2 = jnp.concatenate([ei[0], ei[1]])
    col2 = jnp.concatenate([ei[1], ei[0]])
    col_db = jnp.concatenate(
        [col2, jnp.full((pade - e2,), n, jnp.int32)]).reshape(NC, NS, k, CHUNK)

    blk = NS * CHUNK * grp
    nblk0 = min(max(int(round(SPLIT_F * e2 / blk)), 1), -(-e2 // blk) - 1)
    cap0 = nblk0 * blk
    nblk1 = -(-(e2 - cap0) // blk)
    cap1 = nblk1 * blk
    k0, k1 = nblk0 * grp, nblk1 * grp
    kmax = max(k0, k1)

    def _split(arr, padval):
        a0 = arr[:cap0].reshape(NS, k0, CHUNK)
        a1 = jnp.concatenate(
            [arr[cap0:], jnp.full((cap0 + cap1 - e2,), padval, jnp.int32)]
        ).reshape(NS, k1, CHUNK)
        a0 = jnp.pad(a0, ((0, 0), (0, kmax - k0), (0, 0)))
        a1 = jnp.pad(a1, ((0, 0), (0, kmax - k1), (0, 0)))
        pair = (a0, a1) if SPLIT_SLOW == 0 else (a1, a0)
        return jnp.stack(pair)

    row_r = _split(row2, 0)
    col_r = _split(col2, n)
    kcs = (k0, k1) if SPLIT_SLOW == 0 else (k1, k0)

    ones16 = jnp.ones((CHUNK, DEGW), jnp.float32)
    zdeg = jnp.zeros((rpt, DEGW), jnp.float32)
    zrow = jnp.zeros((rpt, d), jnp.float32)

    mesh = plsc.VectorSubcoreMesh(core_axis_name="c", subcore_axis_name="s",
                                  num_cores=NC, num_subcores=NS)

    deg_call = pl.kernel(
        functools.partial(_sc_deg_body, k, rpt, grp),
        out_type=jax.ShapeDtypeStruct((NC, NS, rpt, DEGW), jnp.float32),
        mesh=mesh,
        scratch_types=[
            pltpu.VMEM((k, CHUNK), jnp.int32),
            pltpu.VMEM((CHUNK, DEGW), jnp.float32),
            pltpu.VMEM_SHARED((np_rows, DEGW), jnp.float32),
            pltpu.SemaphoreType.DMA,
        ],
    )

    prop_call = pl.kernel(
        functools.partial(_sc_prop_body, kcs, rpt, grp),
        out_type=jax.ShapeDtypeStruct((NC, NS, rpt, d), jnp.float32),
        mesh=mesh,
        scratch_types=[
            pltpu.VMEM((grp, CHUNK), jnp.int32),
            pltpu.VMEM((grp, CHUNK), jnp.int32),
            pltpu.VMEM((CHUNK, d), jnp.float32),
            pltpu.VMEM((CHUNK, d), jnp.float32),
            pltpu.VMEM_SHARED((np_rows, d), jnp.float32),
            pltpu.SemaphoreType.DMA,
            pltpu.SemaphoreType.DMA,
        ],
    )

    h1 = _tc(_mm_body, jax.ShapeDtypeStruct((n, d), jnp.float32), x, W1)

    dp = deg_call(col_db, ones16, zdeg)
    dp = dp.reshape(NC, np_rows, DEGW)[:, :n, :]

    dis, g1 = _tc(
        _scale_body,
        [jax.ShapeDtypeStruct((n, 1), jnp.float32),
         jax.ShapeDtypeStruct((n, d), jnp.float32)],
        dp[0], dp[1], h1)

    s1 = prop_call(g1, row_r, col_r, zrow).reshape(NC, np_rows, d)[:, :n, :]
    g2 = _tc(_mid_body, jax.ShapeDtypeStruct((n, d), jnp.float32),
             s1[0], s1[1], g1, dis, W2, b1)

    s2 = prop_call(g2, row_r, col_r, zrow).reshape(NC, np_rows, d)[:, :n, :]
    out = _tc(_out_body, jax.ShapeDtypeStruct((n, d), jnp.float32),
              s2[0], s2[1], g2, dis, b2)
    return out


# speedup vs baseline: 1.0018x; 1.0018x over previous
import functools

import jax
import jax.numpy as jnp
from jax import lax
from jax.experimental import pallas as pl
from jax.experimental.pallas import tpu as pltpu
from jax.experimental.pallas import tpu_sc as plsc

NC = 2
NS = 16
CHUNK = 128
DEGW = 128
SPLIT_SLOW = 1
SPLIT_F = 0.26


def _mm_body(x_ref, w_ref, o_ref):
    o_ref[...] = jnp.dot(x_ref[...], w_ref[...],
                         preferred_element_type=jnp.float32)


def _scale_body(dp0_ref, dp1_ref, h1_ref, dis_ref, g_ref):
    deg = dp0_ref[:, 0:1] + dp1_ref[:, 0:1] + 1.0
    dis = lax.rsqrt(deg)
    dis_ref[...] = dis
    g_ref[...] = dis * h1_ref[...]


def _mid_body(s0_ref, s1_ref, g1_ref, dis_ref, w2_ref, b1_ref, g2_ref):
    st = s0_ref[...] + s1_ref[...] + g1_ref[...]
    conv = dis_ref[...] * st + b1_ref[...]
    a = jnp.maximum(conv, 0.0)
    h2 = jnp.dot(a, w2_ref[...], preferred_element_type=jnp.float32)
    g2_ref[...] = dis_ref[...] * h2


def _out_body(s0_ref, s1_ref, g2_ref, dis_ref, b2_ref, o_ref):
    st = s0_ref[...] + s1_ref[...] + g2_ref[...]
    conv = dis_ref[...] * st + b2_ref[...]
    m = jnp.max(conv, axis=1, keepdims=True)
    e = jnp.exp(conv - m)
    z = jnp.sum(e, axis=1, keepdims=True)
    o_ref[...] = conv - m - jnp.log(z)


def _tc(body, out_shape, *args):
    return pl.pallas_call(body, out_shape=out_shape)(*args)


def _sc_deg_body(nchunks, rpt, grp, col_hbm, ones_hbm, z_hbm, dp_out,
                 colv, ones_v, deg_sp, sems):
    c = lax.axis_index("c")
    s = lax.axis_index("s")
    pltpu.sync_copy(col_hbm.at[c, s], colv)
    pltpu.sync_copy(ones_hbm, ones_v)
    pltpu.sync_copy(z_hbm, deg_sp.at[pl.ds(s * rpt, rpt)])
    plsc.subcore_barrier()

    def group(gi, carry):
        descs = [pltpu.async_copy(ones_v, deg_sp.at[colv.at[gi * grp + b]],
                                  sems, add=True) for b in range(grp)]
        for desc in descs:
            desc.wait()
        return carry

    lax.fori_loop(0, nchunks // grp, group, 0)
    plsc.subcore_barrier()
    pltpu.sync_copy(deg_sp.at[pl.ds(s * rpt, rpt)], dp_out.at[c, s])


def _sc_prop_body(kcs, rpt, grp, g_hbm, row_hbm, col_hbm, z_hbm, sp_out,
                  rowv, colv, gbuf0, gbuf1, acc_sp, semg, sems):
    c = lax.axis_index("c")
    s = lax.axis_index("s")
    pltpu.sync_copy(z_hbm, acc_sp.at[pl.ds(s * rpt, rpt)])
    plsc.subcore_barrier()
    bufs = (gbuf0, gbuf1)

    def group(gi, carry):
        pltpu.sync_copy(row_hbm.at[c, s, pl.ds(gi * grp, grp)], rowv)
        pltpu.sync_copy(col_hbm.at[c, s, pl.ds(gi * grp, grp)], colv)
        g_desc = pltpu.async_copy(g_hbm.at[rowv.at[0]], bufs[0], semg)
        s_desc = None
        for b in range(grp):
            g_desc.wait()
            if s_desc is not None:
                s_desc.wait()
            if b + 1 < grp:
                g_desc = pltpu.async_copy(g_hbm.at[rowv.at[b + 1]],
                                          bufs[(b + 1) % 2], semg)
            s_desc = pltpu.async_copy(bufs[b % 2], acc_sp.at[colv.at[b]],
                                      sems, add=True)
        s_desc.wait()
        return carry

    ngrps = jnp.where(c == 0, kcs[0] // grp, kcs[1] // grp)
    lax.fori_loop(0, ngrps, group, 0)
    plsc.subcore_barrier()
    pltpu.sync_copy(acc_sp.at[pl.ds(s * rpt, rpt)], sp_out.at[c, s])


@jax.jit
def kernel(x, edge_index, W1, b1, W2, b2):
    n, d = x.shape
    e = edge_index.shape[1]
    e2 = 2 * e
    grp = 16
    k = -(-e2 // (NC * NS * CHUNK * grp)) * grp
    pade = NC * NS * k * CHUNK
    np_rows = -(-(n + 1) // (NS * 8)) * (NS * 8)
    rpt = np_rows // NS

    ei = edge_index.astype(jnp.int32)
    row2 = jnp.concatenate([ei[0], ei[1]])
    col2 = jnp.concatenate([ei[1], ei[0]])
    col_db = jnp.concatenate(
        [col2, jnp.full((pade - e2,), n, jnp.int32)]).reshape(NC, NS, k, CHUNK)

    blk = NS * CHUNK * grp
    nblk0 = min(max(int(round(SPLIT_F * e2 / blk)), 1), -(-e2 // blk) - 1)
    cap0 = nblk0 * blk
    nblk1 = -(-(e2 - cap0) // blk)
    cap1 = nblk1 * blk
    k0, k1 = nblk0 * grp, nblk1 * grp
    kmax = max(k0, k1)

    def _split(arr, padval):
        a0 = arr[:cap0].reshape(NS, k0, CHUNK)
        a1 = jnp.concatenate(
            [arr[cap0:], jnp.full((cap0 + cap1 - e2,), padval, jnp.int32)]
        ).reshape(NS, k1, CHUNK)
        a0 = jnp.pad(a0, ((0, 0), (0, kmax - k0), (0, 0)))
        a1 = jnp.pad(a1, ((0, 0), (0, kmax - k1), (0, 0)))
        pair = (a0, a1) if SPLIT_SLOW == 0 else (a1, a0)
        return jnp.stack(pair)

    row_r = _split(row2, 0)
    col_r = _split(col2, n)
    kcs = (k0, k1) if SPLIT_SLOW == 0 else (k1, k0)

    ones16 = jnp.ones((CHUNK, DEGW), jnp.float32)
    zdeg = jnp.zeros((rpt, DEGW), jnp.float32)
    zrow = jnp.zeros((rpt, d), jnp.float32)

    mesh = plsc.VectorSubcoreMesh(core_axis_name="c", subcore_axis_name="s",
                                  num_cores=NC, num_subcores=NS)

    deg_call = pl.kernel(
        functools.partial(_sc_deg_body, k, rpt, grp),
        out_type=jax.ShapeDtypeStruct((NC, NS, rpt, DEGW), jnp.float32),
        mesh=mesh,
        scratch_types=[
            pltpu.VMEM((k, CHUNK), jnp.int32),
            pltpu.VMEM((CHUNK, DEGW), jnp.float32),
            pltpu.VMEM_SHARED((np_rows, DEGW), jnp.float32),
            pltpu.SemaphoreType.DMA,
        ],
    )

    prop_call = pl.kernel(
        functools.partial(_sc_prop_body, kcs, rpt, grp),
        out_type=jax.ShapeDtypeStruct((NC, NS, rpt, d), jnp.float32),
        mesh=mesh,
        scratch_types=[
            pltpu.VMEM((grp, CHUNK), jnp.int32),
            pltpu.VMEM((grp, CHUNK), jnp.int32),
            pltpu.VMEM((CHUNK, d), jnp.float32),
            pltpu.VMEM((CHUNK, d), jnp.float32),
            pltpu.VMEM_SHARED((np_rows, d), jnp.float32),
            pltpu.SemaphoreType.DMA,
            pltpu.SemaphoreType.DMA,
        ],
    )

    h1 = _tc(_mm_body, jax.ShapeDtypeStruct((n, d), jnp.float32), x, W1)

    dp = deg_call(col_db, ones16, zdeg)
    dp = dp.reshape(NC, np_rows, DEGW)[:, :n, :]

    dis, g1 = _tc(
        _scale_body,
        [jax.ShapeDtypeStruct((n, 1), jnp.float32),
         jax.ShapeDtypeStruct((n, d), jnp.float32)],
        dp[0], dp[1], h1)

    s1 = prop_call(g1, row_r, col_r, zrow).reshape(NC, np_rows, d)[:, :n, :]
    g2 = _tc(_mid_body, jax.ShapeDtypeStruct((n, d), jnp.float32),
             s1[0], s1[1], g1, dis, W2, b1)

    s2 = prop_call(g2, row_r, col_r, zrow).reshape(NC, np_rows, d)[:, :n, :]
    out = _tc(_out_body, jax.ShapeDtypeStruct((n, d), jnp.float32),
              s2[0], s2[1], g2, dis, b2)
    return out


# speedup vs baseline: 1.4871x; 1.4845x over previous
import functools

import jax
import jax.numpy as jnp
from jax import lax
from jax.experimental import pallas as pl
from jax.experimental.pallas import tpu as pltpu
from jax.experimental.pallas import tpu_sc as plsc

NC = 2
NS = 16
CHUNK = 96
DEGW = 128
SPLIT_SLOW = 0
SPLIT_F = 0.5


def _mm_body(x_ref, w_ref, o_ref):
    o_ref[...] = jnp.dot(x_ref[...], w_ref[...],
                         preferred_element_type=jnp.float32)


def _scale_body(dp0_ref, dp1_ref, h1_ref, dis_ref, g_ref):
    deg = dp0_ref[:, 0:1] + dp1_ref[:, 0:1] + 1.0
    dis = lax.rsqrt(deg)
    dis_ref[...] = dis
    g_ref[...] = dis * h1_ref[...]


def _mid_body(s0_ref, s1_ref, g1_ref, dis_ref, w2_ref, b1_ref, g2_ref):
    st = s0_ref[...] + s1_ref[...] + g1_ref[...]
    conv = dis_ref[...] * st + b1_ref[...]
    a = jnp.maximum(conv, 0.0)
    h2 = jnp.dot(a, w2_ref[...], preferred_element_type=jnp.float32)
    g2_ref[...] = dis_ref[...] * h2


def _out_body(s0_ref, s1_ref, g2_ref, dis_ref, b2_ref, o_ref):
    st = s0_ref[...] + s1_ref[...] + g2_ref[...]
    conv = dis_ref[...] * st + b2_ref[...]
    m = jnp.max(conv, axis=1, keepdims=True)
    e = jnp.exp(conv - m)
    z = jnp.sum(e, axis=1, keepdims=True)
    o_ref[...] = conv - m - jnp.log(z)


def _tc(body, out_shape, *args):
    return pl.pallas_call(body, out_shape=out_shape)(*args)


def _sc_deg_body(nchunks, rpt, grp, col_hbm, ones_hbm, z_hbm, dp_out,
                 colv, ones_v, deg_sp, sems):
    c = lax.axis_index("c")
    s = lax.axis_index("s")
    pltpu.sync_copy(col_hbm.at[c, s], colv)
    pltpu.sync_copy(ones_hbm, ones_v)
    pltpu.sync_copy(z_hbm, deg_sp.at[pl.ds(s * rpt, rpt)])
    plsc.subcore_barrier()

    def group(gi, carry):
        descs = [pltpu.async_copy(ones_v, deg_sp.at[colv.at[gi * grp + b]],
                                  sems, add=True) for b in range(grp)]
        for desc in descs:
            desc.wait()
        return carry

    lax.fori_loop(0, nchunks // grp, group, 0)
    plsc.subcore_barrier()
    pltpu.sync_copy(deg_sp.at[pl.ds(s * rpt, rpt)], dp_out.at[c, s])


def _sc_prop_body(kcs, rpt, grp, g_hbm, row_hbm, col_hbm, z_hbm, sp_out,
                  rowv, colv, gbuf0, gbuf1, gbuf2,
                  acc_sp, sg0, sg1, sg2, ss0, ss1, ss2):
    c = lax.axis_index("c")
    s = lax.axis_index("s")
    pltpu.sync_copy(z_hbm, acc_sp.at[pl.ds(s * rpt, rpt)])
    plsc.subcore_barrier()
    bufs = (gbuf0, gbuf1, gbuf2)
    sgs = (sg0, sg1, sg2)
    sss = (ss0, ss1, ss2)

    def group(gi, carry):
        pltpu.sync_copy(row_hbm.at[c, s, pl.ds(gi * grp, grp)], rowv)
        pltpu.sync_copy(col_hbm.at[c, s, pl.ds(gi * grp, grp)], colv)
        gds = [None] * grp
        sds = [None] * grp
        gds[0] = pltpu.async_copy(g_hbm.at[rowv.at[0]], bufs[0], sgs[0])
        if grp > 1:
            gds[1] = pltpu.async_copy(g_hbm.at[rowv.at[1]], bufs[1], sgs[1])
        for b in range(grp):
            gds[b].wait()
            if b >= 1 and b + 2 < grp:
                sds[b - 1].wait()
            if b + 2 < grp:
                gds[b + 2] = pltpu.async_copy(g_hbm.at[rowv.at[b + 2]],
                                              bufs[(b + 2) % 3], sgs[(b + 2) % 3])
            sds[b] = pltpu.async_copy(bufs[b % 3], acc_sp.at[colv.at[b]],
                                      sss[b % 3], add=True)
        for b in range(max(0, grp - 3), grp):
            sds[b].wait()
        return carry

    ngrps = jnp.where(c == 0, kcs[0] // grp, kcs[1] // grp)
    lax.fori_loop(0, ngrps, group, 0)
    plsc.subcore_barrier()
    pltpu.sync_copy(acc_sp.at[pl.ds(s * rpt, rpt)], sp_out.at[c, s])


@jax.jit
def kernel(x, edge_index, W1, b1, W2, b2):
    n, d = x.shape
    e = edge_index.shape[1]
    e2 = 2 * e
    grp = 8
    k = -(-e2 // (NC * NS * CHUNK * grp)) * grp
    pade = NC * NS * k * CHUNK
    np_rows = -(-(n + 1) // (NS * 8)) * (NS * 8)
    rpt = np_rows // NS

    ei = edge_index.astype(jnp.int32)
    row2 = jnp.concatenate([ei[0], ei[1]])
    col2 = jnp.concatenate([ei[1], ei[0]])
    col_db = jnp.concatenate(
        [col2, jnp.full((pade - e2,), n, jnp.int32)]).reshape(NC, NS, k, CHUNK)

    blk = NS * CHUNK * grp
    nblk0 = min(max(int(round(SPLIT_F * e2 / blk)), 1), -(-e2 // blk) - 1)
    cap0 = nblk0 * blk
    nblk1 = -(-(e2 - cap0) // blk)
    cap1 = nblk1 * blk
    k0, k1 = nblk0 * grp, nblk1 * grp
    kmax = max(k0, k1)

    def _split(arr, padval):
        a0 = arr[:cap0].reshape(NS, k0, CHUNK)
        a1 = jnp.concatenate(
            [arr[cap0:], jnp.full((cap0 + cap1 - e2,), padval, jnp.int32)]
        ).reshape(NS, k1, CHUNK)
        a0 = jnp.pad(a0, ((0, 0), (0, kmax - k0), (0, 0)))
        a1 = jnp.pad(a1, ((0, 0), (0, kmax - k1), (0, 0)))
        pair = (a0, a1) if SPLIT_SLOW == 0 else (a1, a0)
        return jnp.stack(pair)

    row_r = _split(row2, 0)
    col_r = _split(col2, n)
    kcs = (k0, k1) if SPLIT_SLOW == 0 else (k1, k0)

    ones16 = jnp.ones((CHUNK, DEGW), jnp.float32)
    zdeg = jnp.zeros((rpt, DEGW), jnp.float32)
    zrow = jnp.zeros((rpt, d), jnp.float32)

    mesh = plsc.VectorSubcoreMesh(core_axis_name="c", subcore_axis_name="s",
                                  num_cores=NC, num_subcores=NS)

    deg_call = pl.kernel(
        functools.partial(_sc_deg_body, k, rpt, grp),
        out_type=jax.ShapeDtypeStruct((NC, NS, rpt, DEGW), jnp.float32),
        mesh=mesh,
        scratch_types=[
            pltpu.VMEM((k, CHUNK), jnp.int32),
            pltpu.VMEM((CHUNK, DEGW), jnp.float32),
            pltpu.VMEM_SHARED((np_rows, DEGW), jnp.float32),
            pltpu.SemaphoreType.DMA,
        ],
    )

    prop_call = pl.kernel(
        functools.partial(_sc_prop_body, kcs, rpt, grp),
        out_type=jax.ShapeDtypeStruct((NC, NS, rpt, d), jnp.float32),
        mesh=mesh,
        scratch_types=[
            pltpu.VMEM((grp, CHUNK), jnp.int32),
            pltpu.VMEM((grp, CHUNK), jnp.int32),
            pltpu.VMEM((CHUNK, d), jnp.float32),
            pltpu.VMEM((CHUNK, d), jnp.float32),
            pltpu.VMEM((CHUNK, d), jnp.float32),
            pltpu.VMEM_SHARED((np_rows, d), jnp.float32),
            pltpu.SemaphoreType.DMA,
            pltpu.SemaphoreType.DMA,
            pltpu.SemaphoreType.DMA,
            pltpu.SemaphoreType.DMA,
            pltpu.SemaphoreType.DMA,
            pltpu.SemaphoreType.DMA,
        ],
    )

    h1 = _tc(_mm_body, jax.ShapeDtypeStruct((n, d), jnp.float32), x, W1)

    dp = deg_call(col_db, ones16, zdeg)
    dp = dp.reshape(NC, np_rows, DEGW)[:, :n, :]

    dis, g1 = _tc(
        _scale_body,
        [jax.ShapeDtypeStruct((n, 1), jnp.float32),
         jax.ShapeDtypeStruct((n, d), jnp.float32)],
        dp[0], dp[1], h1)

    s1 = prop_call(g1, row_r, col_r, zrow).reshape(NC, np_rows, d)[:, :n, :]
    g2 = _tc(_mid_body, jax.ShapeDtypeStruct((n, d), jnp.float32),
             s1[0], s1[1], g1, dis, W2, b1)

    s2 = prop_call(g2, row_r, col_r, zrow).reshape(NC, np_rows, d)[:, :n, :]
    out = _tc(_out_body, jax.ShapeDtypeStruct((n, d), jnp.float32),
              s2[0], s2[1], g2, dis, b2)
    return out
